# baseline (speedup 1.0000x reference)
import functools

import jax
import jax.numpy as jnp
from jax.experimental import pallas as pl
from jax.experimental.pallas import tpu as pltpu


def _glu_kernel(x_ref, wgv_ref, wout_ref, o_ref, p_ref, *, rank):
    j = pl.program_id(0)
    slot = jax.lax.rem(j, 2)

    p_ref[slot] = jax.lax.dot_general(
        x_ref[...], wgv_ref[...],
        dimension_numbers=(((1,), (1,)), ((), ())),
        preferred_element_type=jnp.float32,
    )

    p_prev = p_ref[1 - slot]
    h = jax.nn.sigmoid(p_prev[:, :rank]) * p_prev[:, rank:]
    o_val = jax.lax.dot_general(
        h, wout_ref[...],
        dimension_numbers=(((1,), (0,)), ((), ())),
        preferred_element_type=jnp.float32,
    )

    @pl.when(j > 0)
    def _store():
        o_ref[...] = o_val


@functools.partial(jax.jit, static_argnames=())
def kernel(x, W_in, A_log, A_imag, W_dt, W_phase, W_out):
    del A_log, A_imag, W_dt, W_phase
    m, d_model = x.shape
    rank = W_out.shape[1]
    w_gv = W_in[: 2 * rank]
    w_out_t = W_out.T

    tm = 2048
    nb = m // tm
    grid = (nb + 1,)
    return pl.pallas_call(
        functools.partial(_glu_kernel, rank=rank),
        grid=grid,
        in_specs=[
            pl.BlockSpec((tm, d_model), lambda j: (min(j, nb - 1) if isinstance(j, int) else jnp.minimum(j, nb - 1), 0)),
            pl.BlockSpec((2 * rank, d_model), lambda j: (0, 0)),
            pl.BlockSpec((rank, d_model), lambda j: (0, 0)),
        ],
        out_specs=pl.BlockSpec((tm, d_model), lambda j: (jnp.maximum(j - 1, 0), 0)),
        out_shape=jax.ShapeDtypeStruct((m, d_model), jnp.float32),
        scratch_shapes=[pltpu.VMEM((2, tm, 2 * rank), jnp.float32)],
    )(x, w_gv, w_out_t)


# speedup vs baseline: 1.2442x; 1.2442x over previous
import functools

import jax
import jax.numpy as jnp
from jax.experimental import pallas as pl
from jax.experimental.pallas import tpu as pltpu


def _glu_kernel(x_ref, wgv_ref, wout_ref, o_ref, *, rank):
    p = jax.lax.dot_general(
        x_ref[...], wgv_ref[...],
        dimension_numbers=(((1,), (1,)), ((), ())),
        preferred_element_type=jnp.float32,
    )
    h = jax.nn.sigmoid(p[:, :rank]) * p[:, rank:]
    o_ref[...] = jax.lax.dot_general(
        h, wout_ref[...],
        dimension_numbers=(((1,), (0,)), ((), ())),
        preferred_element_type=jnp.float32,
    )


@jax.jit
def kernel(x, W_in, A_log, A_imag, W_dt, W_phase, W_out):
    del A_log, A_imag, W_dt, W_phase
    m, d_model = x.shape
    rank = W_out.shape[1]
    w_gv = W_in[: 2 * rank]
    w_out_t = W_out.T

    tm = 2048
    grid = (m // tm,)
    return pl.pallas_call(
        functools.partial(_glu_kernel, rank=rank),
        grid=grid,
        in_specs=[
            pl.BlockSpec((tm, d_model), lambda j: (j, 0)),
            pl.BlockSpec((2 * rank, d_model), lambda j: (0, 0)),
            pl.BlockSpec((rank, d_model), lambda j: (0, 0)),
        ],
        out_specs=pl.BlockSpec((tm, d_model), lambda j: (j, 0)),
        out_shape=jax.ShapeDtypeStruct((m, d_model), jnp.float32),
        compiler_params=pltpu.CompilerParams(
            dimension_semantics=("parallel",),
        ),
    )(x, w_gv, w_out_t)
